# Initial kernel scaffold; baseline (speedup 1.0000x reference)
import functools

import jax
import jax.numpy as jnp
from jax import lax
from jax.experimental import pallas as pl
from jax.experimental.pallas import tpu as pltpu
from jax.experimental.pallas import tpu_sc as plsc

B = 16384
D = 64
NEG = 20
R = NEG + 1
SLOTS = 32
NC = 2
NS = 16
NW = NC * NS
SPW = B // NW
CS = 32
NCHUNK = SPW // CS
L = 16
CH = D // L


def _sc_scores(pu, idx, utab, vtab):
    mesh = plsc.VectorSubcoreMesh(core_axis_name="c", subcore_axis_name="s")

    @functools.partial(
        pl.kernel,
        out_type=jax.ShapeDtypeStruct((B * SLOTS,), jnp.float32),
        mesh=mesh,
        scratch_types=[
            pltpu.VMEM((CS,), jnp.int32),
            pltpu.VMEM((CS * R,), jnp.int32),
            pltpu.VMEM((CS, D), jnp.float32),
            pltpu.VMEM((CS * R, D), jnp.float32),
            pltpu.VMEM((CS * SLOTS,), jnp.float32),
            pltpu.VMEM((L * L,), jnp.float32),
            pltpu.SemaphoreType.DMA,
            pltpu.SemaphoreType.DMA,
        ],
    )
    def k(pu_hbm, idx_hbm, utab_hbm, vtab_hbm, out_hbm,
          pu_v, idx_v, urows, vrows, scores, tbuf, sem_u, sem_v):
        wid = lax.axis_index("s") * NC + lax.axis_index("c")
        flat_base = lax.iota(jnp.int32, (L,)) * L
        zero16 = jnp.zeros((L,), jnp.float32)

        def chunk_body(ci, carry):
            s0 = wid * SPW + ci * CS
            pltpu.sync_copy(pu_hbm.at[pl.ds(s0, CS)], pu_v)
            pltpu.sync_copy(idx_hbm.at[pl.ds(s0 * R, CS * R)], idx_v)
            cu = pltpu.async_copy(utab_hbm.at[pu_v], urows, sem_u)
            cv = pltpu.async_copy(vtab_hbm.at[idx_v], vrows, sem_v)
            cu.wait()
            cv.wait()

            def sample_body(i, carry2):
                u = [urows[i, pl.ds(c * L, L)] for c in range(CH)]

                def dot_partial(k_):
                    p = u[0] * vrows[i * R + k_, pl.ds(0, L)]
                    for c in range(1, CH):
                        p = p + u[c] * vrows[i * R + k_, pl.ds(c * L, L)]
                    return p

                def col_reduce():
                    acc = zero16
                    for c in range(L):
                        acc = acc + plsc.load_gather(tbuf, [flat_base + c])
                    return acc

                for k_ in range(L):
                    tbuf[pl.ds(k_ * L, L)] = dot_partial(k_)
                scores[pl.ds(i * SLOTS, L)] = col_reduce()
                for k_ in range(L, R):
                    tbuf[pl.ds((k_ - L) * L, L)] = dot_partial(k_)
                for k_ in range(R - L, L):
                    tbuf[pl.ds(k_ * L, L)] = zero16
                scores[pl.ds(i * SLOTS + L, L)] = col_reduce()
                return carry2

            lax.fori_loop(0, CS, sample_body, 0)
            pltpu.sync_copy(scores, out_hbm.at[pl.ds(s0 * SLOTS, CS * SLOTS)])
            return carry

        lax.fori_loop(0, NCHUNK, chunk_body, 0)

    return k(pu, idx, utab, vtab)


def _tc_reduce(x):
    def body(x_ref, o_ref):
        v = x_ref[...]
        lanes = lax.broadcasted_iota(jnp.int32, v.shape, 1)
        valid = (lanes % SLOTS) < R
        v = jnp.clip(v, -10.0, 10.0)
        sp = jnp.where(valid, jax.nn.softplus(v), 0.0)
        o_ref[0, 0] = jnp.sum(sp) / B

    out = pl.pallas_call(
        body,
        out_shape=jax.ShapeDtypeStruct((1, 1), jnp.float32),
        out_specs=pl.BlockSpec(memory_space=pltpu.SMEM),
    )(x)
    return out.reshape(())


def kernel(pos_u, pos_v, neg_v, u_emb_0, v_emb_0):
    pu = pos_u.reshape(B).astype(jnp.int32)
    idx = jnp.concatenate(
        [pos_v.reshape(B, 1), neg_v.reshape(B, NEG)], axis=1
    ).astype(jnp.int32).reshape(B * R)
    scores = _sc_scores(pu, idx, u_emb_0, v_emb_0)
    return _tc_reduce(scores.reshape(B * SLOTS // 128, 128))


# speedup vs baseline: 4.8654x; 4.8654x over previous
import functools

import jax
import jax.numpy as jnp
from jax import lax
from jax.experimental import pallas as pl
from jax.experimental.pallas import tpu as pltpu
from jax.experimental.pallas import tpu_sc as plsc

B = 16384
D = 64
NEG = 20
R = NEG + 1
SLOTS = 32
NC = 2
NS = 16
NW = NC * NS
SPW = B // NW
CS = 32
NCHUNK = SPW // CS
L = 16
CH = D // L


def _sc_scores(pu, idx, utab, vtab):
    mesh = plsc.VectorSubcoreMesh(core_axis_name="c", subcore_axis_name="s")

    @functools.partial(
        pl.kernel,
        out_type=jax.ShapeDtypeStruct((B * SLOTS,), jnp.float32),
        mesh=mesh,
        compiler_params=pltpu.CompilerParams(
            needs_layout_passes=False, use_tc_tiling_on_sc=False),
        scratch_types=[
            pltpu.VMEM((CS,), jnp.int32),
            pltpu.VMEM((CS * R,), jnp.int32),
            pltpu.VMEM((CS, D), jnp.float32),
            pltpu.VMEM((CS * R, D), jnp.float32),
            pltpu.VMEM((CS * SLOTS,), jnp.float32),
            pltpu.VMEM((L * L,), jnp.float32),
            pltpu.SemaphoreType.DMA,
            pltpu.SemaphoreType.DMA,
        ],
    )
    def k(pu_hbm, idx_hbm, utab_hbm, vtab_hbm, out_hbm,
          pu_v, idx_v, urows, vrows, scores, tbuf, sem_u, sem_v):
        wid = lax.axis_index("s") * NC + lax.axis_index("c")
        flat_base = lax.iota(jnp.int32, L) * L
        zero16 = jnp.zeros((L,), jnp.float32)

        def chunk_body(ci, carry):
            s0 = wid * SPW + ci * CS
            pltpu.sync_copy(pu_hbm.at[pl.ds(s0, CS)], pu_v)
            pltpu.sync_copy(idx_hbm.at[pl.ds(s0 * R, CS * R)], idx_v)
            cu = pltpu.async_copy(utab_hbm.at[pu_v], urows, sem_u)
            cv = pltpu.async_copy(vtab_hbm.at[idx_v], vrows, sem_v)
            cu.wait()
            cv.wait()

            def sample_body(i, carry2):
                u = [urows[i, pl.ds(c * L, L)] for c in range(CH)]

                def dot_partial(k_):
                    p = u[0] * vrows[i * R + k_, pl.ds(0, L)]
                    for c in range(1, CH):
                        p = p + u[c] * vrows[i * R + k_, pl.ds(c * L, L)]
                    return p

                def col_reduce():
                    acc = zero16
                    for c in range(L):
                        acc = acc + plsc.load_gather(tbuf, [flat_base + c])
                    return acc

                for k_ in range(L):
                    tbuf[pl.ds(k_ * L, L)] = dot_partial(k_)
                scores[pl.ds(i * SLOTS, L)] = col_reduce()
                for k_ in range(L, R):
                    tbuf[pl.ds((k_ - L) * L, L)] = dot_partial(k_)
                for k_ in range(R - L, L):
                    tbuf[pl.ds(k_ * L, L)] = zero16
                scores[pl.ds(i * SLOTS + L, L)] = col_reduce()
                return carry2

            lax.fori_loop(0, CS, sample_body, 0)
            pltpu.sync_copy(scores, out_hbm.at[pl.ds(s0 * SLOTS, CS * SLOTS)])
            return carry

        lax.fori_loop(0, NCHUNK, chunk_body, 0)

    return k(pu, idx, utab, vtab)


def _tc_reduce(x):
    def body(x_ref, o_ref):
        v = x_ref[...]
        lanes = lax.broadcasted_iota(jnp.int32, v.shape, 1)
        valid = (lanes % SLOTS) < R
        v = jnp.clip(v, -10.0, 10.0)
        sp = jnp.where(valid, jax.nn.softplus(v), 0.0)
        o_ref[0, 0] = jnp.sum(sp) / B

    out = pl.pallas_call(
        body,
        out_shape=jax.ShapeDtypeStruct((1, 1), jnp.float32),
        out_specs=pl.BlockSpec(memory_space=pltpu.SMEM),
    )(x)
    return out.reshape(())


def kernel(pos_u, pos_v, neg_v, u_emb_0, v_emb_0):
    pu = pos_u.reshape(B).astype(jnp.int32)
    idx = jnp.concatenate(
        [pos_v.reshape(B, 1), neg_v.reshape(B, NEG)], axis=1
    ).astype(jnp.int32).reshape(B * R)
    scores = _sc_scores(pu, idx, u_emb_0, v_emb_0)
    return _tc_reduce(scores.reshape(B * SLOTS // 128, 128))
